# baseline (speedup 1.0000x reference)
import functools

import jax
import jax.numpy as jnp
from jax import lax
from jax.experimental import pallas as pl
from jax.experimental.pallas import tpu as pltpu
from jax.experimental.pallas import tpu_sc as plsc

N = 10000
D = 256
F = 16
C = 7
E = 160000

NC = 2
NS = 16
NW = NC * NS
CHUNK = 128
CPS = 40
E_ROWS = E // CHUNK
REAL_ROWS_LAST = E_ROWS - (NW - 1) * CPS
PAD_ROWS = CPS - REAL_ROWS_LAST
N_PAD = 10240
RPS = N_PAD // NS
NB = 8
GB = 5


def _sc_mesh():
    return plsc.VectorSubcoreMesh(core_axis_name="c", subcore_axis_name="s")


_SC_PARAMS = pltpu.CompilerParams(use_tc_tiling_on_sc=False)


def _load_idx(idx_hbm, pad_hbm, idx_vmem, wid):
    @pl.when(wid < NW - 1)
    def _():
        pltpu.sync_copy(idx_hbm.at[pl.ds(wid * CPS, CPS)], idx_vmem)

    @pl.when(wid == NW - 1)
    def _():
        pltpu.sync_copy(idx_hbm.at[pl.ds((NW - 1) * CPS, REAL_ROWS_LAST)],
                        idx_vmem.at[pl.ds(0, REAL_ROWS_LAST)])
        pltpu.sync_copy(pad_hbm, idx_vmem.at[pl.ds(REAL_ROWS_LAST, PAD_ROWS)])


def _sc_hist(dst2d, dpad, ones, zeros):
    @functools.partial(
        pl.kernel,
        out_type=jax.ShapeDtypeStruct((NC, N_PAD, F), jnp.float32),
        mesh=_sc_mesh(),
        scratch_types=[
            pltpu.VMEM((CPS, CHUNK), jnp.int32),
            pltpu.VMEM((CHUNK, F), jnp.float32),
            pltpu.VMEM_SHARED((N_PAD, F), jnp.float32),
            pltpu.SemaphoreType.DMA,
        ],
        compiler_params=_SC_PARAMS,
    )
    def k(dst_hbm, dpad_hbm, ones_hbm, z_hbm, out_hbm, didx, obuf, acc, hsem):
        cid = lax.axis_index("c")
        sid = lax.axis_index("s")
        wid = cid * NS + sid
        pltpu.sync_copy(z_hbm, acc.at[pl.ds(sid * RPS, RPS)])
        pltpu.sync_copy(ones_hbm, obuf)
        _load_idx(dst_hbm, dpad_hbm, didx, wid)
        plsc.subcore_barrier()
        for k in range(NB):
            pltpu.async_copy(obuf, acc.at[didx.at[k]], hsem, add=True)

        @pl.loop(NB, CPS)
        def _(j):
            pltpu.make_async_copy(obuf, acc.at[didx.at[j - NB]], hsem).wait()
            pltpu.async_copy(obuf, acc.at[didx.at[j]], hsem, add=True)

        @pl.loop(CPS - NB, CPS)
        def _(j):
            pltpu.make_async_copy(obuf, acc.at[didx.at[j]], hsem).wait()

        plsc.subcore_barrier()
        pltpu.sync_copy(acc.at[pl.ds(sid * RPS, RPS)],
                        out_hbm.at[cid].at[pl.ds(sid * RPS, RPS)])

    return k(dst2d, dpad, ones, zeros)


def _sc_segsum(g, src2d, dst2d, spad, dpad, zeros):
    @functools.partial(
        pl.kernel,
        out_type=jax.ShapeDtypeStruct((NC, N_PAD, F), jnp.float32),
        mesh=_sc_mesh(),
        scratch_types=[
            pltpu.VMEM((CPS, CHUNK), jnp.int32),
            pltpu.VMEM((CPS, CHUNK), jnp.int32),
            [pltpu.VMEM((CHUNK, F), jnp.float32)] * NB,
            pltpu.VMEM_SHARED((N_PAD, F), jnp.float32),
            [pltpu.SemaphoreType.DMA] * NB,
            [pltpu.SemaphoreType.DMA] * NB,
        ],
        compiler_params=_SC_PARAMS,
    )
    def k(g_hbm, src_hbm, dst_hbm, spad_hbm, dpad_hbm, z_hbm, out_hbm,
          sidx, didx, rows, acc, gsem, ssem):
        cid = lax.axis_index("c")
        sid = lax.axis_index("s")
        wid = cid * NS + sid
        pltpu.sync_copy(z_hbm, acc.at[pl.ds(sid * RPS, RPS)])
        _load_idx(src_hbm, spad_hbm, sidx, wid)
        _load_idx(dst_hbm, dpad_hbm, didx, wid)
        plsc.subcore_barrier()
        for k in range(NB):
            pltpu.async_copy(g_hbm.at[sidx.at[k]], rows[k], gsem[k])

        @pl.loop(0, CPS, step=NB)
        def _(j):
            for k in range(NB):
                pltpu.make_async_copy(g_hbm.at[sidx.at[j + k]],
                                      rows[k], gsem[k]).wait()
                pltpu.async_copy(rows[k], acc.at[didx.at[j + k]], ssem[k],
                                 add=True)
            for k in range(NB):
                pltpu.make_async_copy(rows[k], acc.at[didx.at[j + k]],
                                      ssem[k]).wait()

                @pl.when(j + k + NB < CPS)
                def _():
                    pltpu.async_copy(g_hbm.at[sidx.at[j + k + NB]],
                                     rows[k], gsem[k])

        plsc.subcore_barrier()
        pltpu.sync_copy(acc.at[pl.ds(sid * RPS, RPS)],
                        out_hbm.at[cid].at[pl.ds(sid * RPS, RPS)])

    return k(g, src2d, dst2d, spad, dpad, zeros)


def _tc_l1(counts, x, W1):
    def body(c_ref, x_ref, w_ref, dis_ref, g_ref):
        deg = c_ref[0] + c_ref[1] + 1.0
        dis = 1.0 / jnp.sqrt(deg)
        dis_ref[...] = dis
        h1 = jnp.dot(x_ref[...], w_ref[...],
                     preferred_element_type=jnp.float32,
                     precision=lax.Precision.HIGHEST)
        g_ref[...] = h1 * dis

    B = N // GB
    return pl.pallas_call(
        body,
        out_shape=(jax.ShapeDtypeStruct((N, F), jnp.float32),
                   jax.ShapeDtypeStruct((N, F), jnp.float32)),
        grid=(GB,),
        in_specs=[pl.BlockSpec((2, B, F), lambda i: (0, i, 0)),
                  pl.BlockSpec((B, D), lambda i: (i, 0)),
                  pl.BlockSpec((D, F), lambda i: (0, 0))],
        out_specs=(pl.BlockSpec((B, F), lambda i: (i, 0)),
                   pl.BlockSpec((B, F), lambda i: (i, 0))),
    )(counts, x, W1)


def _tc_mid(s1, g1, dis, b1, W2p):
    def body(s_ref, g_ref, d_ref, b_ref, w_ref, h1_ref, g2_ref):
        pre = d_ref[...] * (s_ref[0] + s_ref[1] + g_ref[...]) + b_ref[...]
        H1 = jnp.maximum(pre, 0.0)
        h1_ref[...] = H1
        h2 = jnp.dot(H1, w_ref[...], preferred_element_type=jnp.float32,
                     precision=lax.Precision.HIGHEST)
        g2_ref[...] = h2 * d_ref[...]

    B = N // GB
    return pl.pallas_call(
        body,
        out_shape=(jax.ShapeDtypeStruct((N, F), jnp.float32),
                   jax.ShapeDtypeStruct((N, F), jnp.float32)),
        grid=(GB,),
        in_specs=[pl.BlockSpec((2, B, F), lambda i: (0, i, 0)),
                  pl.BlockSpec((B, F), lambda i: (i, 0)),
                  pl.BlockSpec((B, F), lambda i: (i, 0)),
                  pl.BlockSpec((1, F), lambda i: (0, 0)),
                  pl.BlockSpec((F, F), lambda i: (0, 0))],
        out_specs=(pl.BlockSpec((B, F), lambda i: (i, 0)),
                   pl.BlockSpec((B, F), lambda i: (i, 0))),
    )(s1, g1, dis, b1, W2p)


def _tc_post(s2, g2, dis, b2p):
    def body(s_ref, g_ref, d_ref, b_ref, h2_ref, lp_ref):
        t = d_ref[...] * (s_ref[0] + s_ref[1] + g_ref[...]) + b_ref[...]
        h2_ref[...] = t
        r = jnp.maximum(t, 0.0)
        col = lax.broadcasted_iota(jnp.int32, r.shape, 1)
        rm = jnp.where(col < C, r, -jnp.inf)
        m = jnp.max(rm, axis=1, keepdims=True)
        lse = m + jnp.log(jnp.sum(jnp.exp(rm - m), axis=1, keepdims=True))
        lp_ref[...] = r - lse

    B = N // GB
    return pl.pallas_call(
        body,
        out_shape=(jax.ShapeDtypeStruct((N, F), jnp.float32),
                   jax.ShapeDtypeStruct((N, F), jnp.float32)),
        grid=(GB,),
        in_specs=[pl.BlockSpec((2, B, F), lambda i: (0, i, 0)),
                  pl.BlockSpec((B, F), lambda i: (i, 0)),
                  pl.BlockSpec((B, F), lambda i: (i, 0)),
                  pl.BlockSpec((1, F), lambda i: (0, 0))],
        out_specs=(pl.BlockSpec((B, F), lambda i: (i, 0)),
                   pl.BlockSpec((B, F), lambda i: (i, 0))),
    )(s2, g2, dis, b2p)


def kernel(x, edge_index, W1, b1, W2, b2):
    ei = edge_index.astype(jnp.int32)
    src2d = ei[0].reshape(E_ROWS, CHUNK)
    dst2d = ei[1].reshape(E_ROWS, CHUNK)
    spad = jnp.zeros((PAD_ROWS, CHUNK), jnp.int32)
    dpad = jnp.full((PAD_ROWS, CHUNK), N, jnp.int32)
    zeros = jnp.zeros((RPS, F), jnp.float32)
    ones = jnp.ones((CHUNK, F), jnp.float32)
    W2p = jnp.pad(W2, ((0, 0), (0, F - C)))
    b1r = b1.reshape(1, F)
    b2p = jnp.pad(b2, (0, F - C)).reshape(1, F)

    counts = _sc_hist(dst2d, dpad, ones, zeros)
    dis, g1 = _tc_l1(counts, x, W1)
    s1 = _sc_segsum(g1, src2d, dst2d, spad, dpad, zeros)
    H1, g2 = _tc_mid(s1, g1, dis, b1r, W2p)
    s2 = _sc_segsum(g2, src2d, dst2d, spad, dpad, zeros)
    H2p, lp = _tc_post(s2, g2, dis, b2p)
    return (lp[:, :C], x, H1, H2p[:, :C])
